# Initial kernel scaffold; baseline (speedup 1.0000x reference)
import jax
import jax.numpy as jnp
from jax.experimental import pallas as pl


def kernel(x_c, edge_index_c, batch_c, x_s, edge_index_s, batch_s, W1_c, b1_c, g1_c, be1_c, W2_c, b2_c, g2_c, be2_c, W1_s, b1_s, g1_s, be1_s, W2_s, b2_s, g2_s, be2_s, Wf1, bf1, Wf2, bf2):
    raise NotImplementedError("write your pallas kernel here")


# speedup vs baseline: 23.5472x; 23.5472x over previous
import jax
import jax.numpy as jnp
from jax import lax
from jax.experimental import pallas as pl
from jax.experimental.pallas import tpu as pltpu
from jax.experimental.pallas import tpu_sc as plsc

N = 10000
E = 320000
G = 256
F_IN = 128
H = 64
OUT = 2

NC = 2
NS = 16
CH = 128
N_P = 10240
ROWS = N_P // NS
K_E = -(-E // (NS * CH))
E_P = NS * K_E * CH
K_B = N_P // (NS * CH)
G_P = 264
DW = 8
EPS = 1e-5

_MESH = plsc.VectorSubcoreMesh(
    core_axis_name="core", subcore_axis_name="sub",
    num_cores=NC, num_subcores=NS)

_SC_PARAMS = pltpu.CompilerParams(use_tc_tiling_on_sc=False)

_f32 = jnp.float32


def _sds(shape):
    return jax.ShapeDtypeStruct(shape, _f32)


def _sc_degcnt_body(dst_c3, dst_s3, bat_c3, bat_s3, ones_h, zeros_h,
                    deg_c_o, deg_s_o, cnt_c_o, cnt_s_o,
                    acc_deg, acc_cnt, idx_e, idx_b, ones_v):
    cid = lax.axis_index("core")
    sid = lax.axis_index("sub")

    pltpu.sync_copy(zeros_h.at[pl.ds(sid * ROWS, ROWS)],
                    acc_deg.at[pl.ds(sid * ROWS, ROWS)])

    @pl.when(sid == 0)
    def _():
        pltpu.sync_copy(zeros_h.at[pl.ds(0, G_P)], acc_cnt)

    pltpu.sync_copy(ones_h, ones_v)

    @pl.when(cid == 0)
    def _():
        pltpu.sync_copy(dst_c3.at[sid], idx_e)
        pltpu.sync_copy(bat_c3.at[sid], idx_b)

    @pl.when(cid == 1)
    def _():
        pltpu.sync_copy(dst_s3.at[sid], idx_e)
        pltpu.sync_copy(bat_s3.at[sid], idx_b)

    plsc.subcore_barrier()

    def edge_body(j, c):
        pltpu.sync_copy(ones_v, acc_deg.at[idx_e.at[j]], add=True)
        return c

    lax.fori_loop(0, K_E, edge_body, 0)

    def bat_body(j, c):
        pltpu.sync_copy(ones_v, acc_cnt.at[idx_b.at[j]], add=True)
        return c

    lax.fori_loop(0, K_B, bat_body, 0)

    plsc.subcore_barrier()

    @pl.when(cid == 0)
    def _():
        pltpu.sync_copy(acc_deg.at[pl.ds(sid * ROWS, ROWS)],
                        deg_c_o.at[pl.ds(sid * ROWS, ROWS)])

        @pl.when(sid == 0)
        def _():
            pltpu.sync_copy(acc_cnt, cnt_c_o)

    @pl.when(cid == 1)
    def _():
        pltpu.sync_copy(acc_deg.at[pl.ds(sid * ROWS, ROWS)],
                        deg_s_o.at[pl.ds(sid * ROWS, ROWS)])

        @pl.when(sid == 0)
        def _():
            pltpu.sync_copy(acc_cnt, cnt_s_o)


_sc_degcnt = pl.kernel(
    _sc_degcnt_body,
    out_type=(_sds((N_P, DW)), _sds((N_P, DW)), _sds((G_P, DW)), _sds((G_P, DW))),
    mesh=_MESH,
    compiler_params=_SC_PARAMS,
    scratch_types=[
        pltpu.VMEM_SHARED((N_P, DW), _f32),
        pltpu.VMEM_SHARED((G_P, DW), _f32),
        pltpu.VMEM((K_E, CH), jnp.int32),
        pltpu.VMEM((K_B, CH), jnp.int32),
        pltpu.VMEM((CH, DW), _f32),
    ],
)


def _sc_agg_body(y_c, y_s, src_c3, dst_c3, src_s3, dst_s3,
                 agg_c_o, agg_s_o,
                 acc, src_v, dst_v, rows_v, sem):
    cid = lax.axis_index("core")
    sid = lax.axis_index("sub")

    @pl.when(cid == 0)
    def _():
        pltpu.sync_copy(y_c.at[pl.ds(sid * ROWS, ROWS)],
                        acc.at[pl.ds(sid * ROWS, ROWS)])
        pltpu.sync_copy(src_c3.at[sid], src_v)
        pltpu.sync_copy(dst_c3.at[sid], dst_v)

    @pl.when(cid == 1)
    def _():
        pltpu.sync_copy(y_s.at[pl.ds(sid * ROWS, ROWS)],
                        acc.at[pl.ds(sid * ROWS, ROWS)])
        pltpu.sync_copy(src_s3.at[sid], src_v)
        pltpu.sync_copy(dst_s3.at[sid], dst_v)

    plsc.subcore_barrier()

    def run(y_ref):
        def body(j, c):
            pltpu.async_copy(y_ref.at[src_v.at[j]], rows_v, sem).wait()
            pltpu.sync_copy(rows_v, acc.at[dst_v.at[j]], add=True)
            return c
        lax.fori_loop(0, K_E, body, 0)

    @pl.when(cid == 0)
    def _():
        run(y_c)

    @pl.when(cid == 1)
    def _():
        run(y_s)

    plsc.subcore_barrier()

    @pl.when(cid == 0)
    def _():
        pltpu.sync_copy(acc.at[pl.ds(sid * ROWS, ROWS)],
                        agg_c_o.at[pl.ds(sid * ROWS, ROWS)])

    @pl.when(cid == 1)
    def _():
        pltpu.sync_copy(acc.at[pl.ds(sid * ROWS, ROWS)],
                        agg_s_o.at[pl.ds(sid * ROWS, ROWS)])


_sc_agg = pl.kernel(
    _sc_agg_body,
    out_type=(_sds((N_P, H)), _sds((N_P, H))),
    mesh=_MESH,
    compiler_params=_SC_PARAMS,
    scratch_types=[
        pltpu.VMEM_SHARED((N_P, H), _f32),
        pltpu.VMEM((K_E, CH), jnp.int32),
        pltpu.VMEM((K_E, CH), jnp.int32),
        pltpu.VMEM((CH, H), _f32),
        pltpu.SemaphoreType.DMA,
    ],
)


def _sc_pool_body(h_c, h_s, bat_c3, bat_s3, zeros_h,
                  pool_c_o, pool_s_o,
                  accp, idx_b, rows_v):
    cid = lax.axis_index("core")
    sid = lax.axis_index("sub")

    @pl.when(sid == 0)
    def _():
        pltpu.sync_copy(zeros_h, accp)

    @pl.when(cid == 0)
    def _():
        pltpu.sync_copy(h_c.at[pl.ds(sid * ROWS, ROWS)], rows_v)
        pltpu.sync_copy(bat_c3.at[sid], idx_b)

    @pl.when(cid == 1)
    def _():
        pltpu.sync_copy(h_s.at[pl.ds(sid * ROWS, ROWS)], rows_v)
        pltpu.sync_copy(bat_s3.at[sid], idx_b)

    plsc.subcore_barrier()

    def body(j, c):
        pltpu.sync_copy(rows_v.at[pl.ds(j * CH, CH)],
                        accp.at[idx_b.at[j]], add=True)
        return c

    lax.fori_loop(0, K_B, body, 0)

    plsc.subcore_barrier()

    @pl.when((cid == 0) & (sid == 0))
    def _():
        pltpu.sync_copy(accp, pool_c_o)

    @pl.when((cid == 1) & (sid == 0))
    def _():
        pltpu.sync_copy(accp, pool_s_o)


_sc_pool = pl.kernel(
    _sc_pool_body,
    out_type=(_sds((G_P, H)), _sds((G_P, H))),
    mesh=_MESH,
    compiler_params=_SC_PARAMS,
    scratch_types=[
        pltpu.VMEM_SHARED((G_P, H), _f32),
        pltpu.VMEM((K_B, CH), jnp.int32),
        pltpu.VMEM((ROWS, H), _f32),
    ],
)


def _tc_scale_body(deg_c_r, x_c_r, w_c_r, deg_s_r, x_s_r, w_s_r,
                   y_c_o, dinv_c_o, y_s_o, dinv_s_o):
    for deg_r, x_r, w_r, y_o, dinv_o in (
            (deg_c_r, x_c_r, w_c_r, y_c_o, dinv_c_o),
            (deg_s_r, x_s_r, w_s_r, y_s_o, dinv_s_o)):
        dinv = lax.rsqrt(deg_r[:, 0:1] + 1.0)
        dinv_o[...] = dinv
        y_o[...] = jnp.dot(x_r[...], w_r[...],
                           preferred_element_type=_f32) * dinv


_tc_scale = pl.pallas_call(
    _tc_scale_body,
    out_shape=(_sds((N_P, H)), _sds((N_P, 1)), _sds((N_P, H)), _sds((N_P, 1))),
)


def _bn_relu(z, g, be):
    zr = z[:N]
    mu = jnp.mean(zr, axis=0, keepdims=True)
    var = jnp.mean((zr - mu) ** 2, axis=0, keepdims=True)
    return jnp.maximum((z - mu) * lax.rsqrt(var + EPS) * g + be, 0.0)


def _tc_mid_body(agg_c_r, dinv_c_r, g_c_r, be_c_r, w_c_r,
                 agg_s_r, dinv_s_r, g_s_r, be_s_r, w_s_r,
                 y2_c_o, y2_s_o):
    for agg_r, dinv_r, g_r, be_r, w_r, y2_o in (
            (agg_c_r, dinv_c_r, g_c_r, be_c_r, w_c_r, y2_c_o),
            (agg_s_r, dinv_s_r, g_s_r, be_s_r, w_s_r, y2_s_o)):
        dinv = dinv_r[...]
        h = _bn_relu(agg_r[...] * dinv, g_r[...], be_r[...])
        y2_o[...] = jnp.dot(h, w_r[...], preferred_element_type=_f32) * dinv


_tc_mid = pl.pallas_call(
    _tc_mid_body,
    out_shape=(_sds((N_P, H)), _sds((N_P, H))),
)


def _tc_h2_body(agg_c_r, dinv_c_r, g_c_r, be_c_r,
                agg_s_r, dinv_s_r, g_s_r, be_s_r,
                h_c_o, h_s_o):
    h_c_o[...] = _bn_relu(agg_c_r[...] * dinv_c_r[...], g_c_r[...], be_c_r[...])
    h_s_o[...] = _bn_relu(agg_s_r[...] * dinv_s_r[...], g_s_r[...], be_s_r[...])


_tc_h2 = pl.pallas_call(
    _tc_h2_body,
    out_shape=(_sds((N_P, H)), _sds((N_P, H))),
)


def _tc_head_body(pool_c_r, cnt_c_r, pool_s_r, cnt_s_r,
                  wf1_r, bf1_r, wf2_r, bf2_r, out_o):
    p_c = pool_c_r[0:G] / jnp.maximum(cnt_c_r[0:G, 0:1], 1.0)
    p_s = pool_s_r[0:G] / jnp.maximum(cnt_s_r[0:G, 0:1], 1.0)
    xcat = jnp.concatenate([p_c, p_s], axis=1)
    hh = jnp.maximum(
        jnp.dot(xcat, wf1_r[...], preferred_element_type=_f32) + bf1_r[...],
        0.0)
    out_o[...] = jnp.dot(hh, wf2_r[...], preferred_element_type=_f32) + bf2_r[...]


_tc_head = pl.pallas_call(
    _tc_head_body,
    out_shape=_sds((G, OUT)),
)


def kernel(x_c, edge_index_c, batch_c, x_s, edge_index_s, batch_s,
           W1_c, b1_c, g1_c, be1_c, W2_c, b2_c, g2_c, be2_c,
           W1_s, b1_s, g1_s, be1_s, W2_s, b2_s, g2_s, be2_s,
           Wf1, bf1, Wf2, bf2):
    x_cp = jnp.pad(x_c, ((0, N_P - N), (0, 0)))
    x_sp = jnp.pad(x_s, ((0, N_P - N), (0, 0)))

    def edges3(ei):
        src = jnp.pad(ei[0], (0, E_P - E)).reshape(NS, K_E, CH)
        dst = jnp.pad(ei[1], (0, E_P - E), constant_values=N).reshape(NS, K_E, CH)
        return src, dst

    src_c3, dst_c3 = edges3(edge_index_c)
    src_s3, dst_s3 = edges3(edge_index_s)
    bat_c3 = jnp.pad(batch_c, (0, N_P - N), constant_values=G).reshape(NS, K_B, CH)
    bat_s3 = jnp.pad(batch_s, (0, N_P - N), constant_values=G).reshape(NS, K_B, CH)

    ones_h = jnp.ones((CH, DW), _f32)
    zeros_deg = jnp.zeros((N_P, DW), _f32)
    zeros_pool = jnp.zeros((G_P, H), _f32)

    deg_c, deg_s, cnt_c, cnt_s = _sc_degcnt(
        dst_c3, dst_s3, bat_c3, bat_s3, ones_h, zeros_deg)

    y1_c, dinv_c, y1_s, dinv_s = _tc_scale(
        deg_c, x_cp, W1_c, deg_s, x_sp, W1_s)

    agg1_c, agg1_s = _sc_agg(y1_c, y1_s, src_c3, dst_c3, src_s3, dst_s3)

    g1_c2, be1_c2 = g1_c.reshape(1, H), be1_c.reshape(1, H)
    g1_s2, be1_s2 = g1_s.reshape(1, H), be1_s.reshape(1, H)
    g2_c2, be2_c2 = g2_c.reshape(1, H), be2_c.reshape(1, H)
    g2_s2, be2_s2 = g2_s.reshape(1, H), be2_s.reshape(1, H)

    y2_c, y2_s = _tc_mid(
        agg1_c, dinv_c, g1_c2, be1_c2, W2_c,
        agg1_s, dinv_s, g1_s2, be1_s2, W2_s)

    agg2_c, agg2_s = _sc_agg(y2_c, y2_s, src_c3, dst_c3, src_s3, dst_s3)

    h2_c, h2_s = _tc_h2(
        agg2_c, dinv_c, g2_c2, be2_c2,
        agg2_s, dinv_s, g2_s2, be2_s2)

    pool_c, pool_s = _sc_pool(h2_c, h2_s, bat_c3, bat_s3, zeros_pool)

    return _tc_head(pool_c, cnt_c, pool_s, cnt_s,
                    Wf1, bf1.reshape(1, H), Wf2, bf2.reshape(1, OUT))


# speedup vs baseline: 26.2761x; 1.1159x over previous
import jax
import jax.numpy as jnp
from jax import lax
from jax.experimental import pallas as pl
from jax.experimental.pallas import tpu as pltpu
from jax.experimental.pallas import tpu_sc as plsc

N = 10000
E = 320000
G = 256
F_IN = 128
H = 64
OUT = 2

NC = 2
NS = 16
CH = 128
N_P = 10240
ROWS = N_P // NS
K_E = 158
K_G = K_E + 2
E_P = NS * K_E * CH
K_B = N_P // (NS * CH)
G_P = 264
DW = 8
EPS = 1e-5

_MESH = plsc.VectorSubcoreMesh(
    core_axis_name="core", subcore_axis_name="sub",
    num_cores=NC, num_subcores=NS)

_SC_PARAMS = pltpu.CompilerParams(use_tc_tiling_on_sc=False)

_f32 = jnp.float32


def _sds(shape):
    return jax.ShapeDtypeStruct(shape, _f32)


def _sc_degcnt_body(dst_c3, dst_s3, bat_c3, bat_s3, ones_h, zeros_h,
                    deg_c_o, deg_s_o, cnt_c_o, cnt_s_o,
                    acc_deg, acc_cnt, idx_e, idx_b, ones_v):
    cid = lax.axis_index("core")
    sid = lax.axis_index("sub")

    pltpu.sync_copy(zeros_h.at[pl.ds(sid * ROWS, ROWS)],
                    acc_deg.at[pl.ds(sid * ROWS, ROWS)])

    @pl.when(sid == 0)
    def _():
        pltpu.sync_copy(zeros_h.at[pl.ds(0, G_P)], acc_cnt)

    pltpu.sync_copy(ones_h, ones_v)

    @pl.when(cid == 0)
    def _():
        pltpu.sync_copy(dst_c3.at[sid], idx_e)
        pltpu.sync_copy(bat_c3.at[sid], idx_b)

    @pl.when(cid == 1)
    def _():
        pltpu.sync_copy(dst_s3.at[sid], idx_e)
        pltpu.sync_copy(bat_s3.at[sid], idx_b)

    plsc.subcore_barrier()

    def edge_body(j, c):
        pltpu.sync_copy(ones_v, acc_deg.at[idx_e.at[j]], add=True)
        return c

    lax.fori_loop(0, K_E, edge_body, 0)

    def bat_body(j, c):
        pltpu.sync_copy(ones_v, acc_cnt.at[idx_b.at[j]], add=True)
        return c

    lax.fori_loop(0, K_B, bat_body, 0)

    plsc.subcore_barrier()

    @pl.when(cid == 0)
    def _():
        pltpu.sync_copy(acc_deg.at[pl.ds(sid * ROWS, ROWS)],
                        deg_c_o.at[pl.ds(sid * ROWS, ROWS)])

        @pl.when(sid == 0)
        def _():
            pltpu.sync_copy(acc_cnt, cnt_c_o)

    @pl.when(cid == 1)
    def _():
        pltpu.sync_copy(acc_deg.at[pl.ds(sid * ROWS, ROWS)],
                        deg_s_o.at[pl.ds(sid * ROWS, ROWS)])

        @pl.when(sid == 0)
        def _():
            pltpu.sync_copy(acc_cnt, cnt_s_o)


_sc_degcnt = pl.kernel(
    _sc_degcnt_body,
    out_type=(_sds((N_P, DW)), _sds((N_P, DW)), _sds((G_P, DW)), _sds((G_P, DW))),
    mesh=_MESH,
    compiler_params=_SC_PARAMS,
    scratch_types=[
        pltpu.VMEM_SHARED((N_P, DW), _f32),
        pltpu.VMEM_SHARED((G_P, DW), _f32),
        pltpu.VMEM((K_E, CH), jnp.int32),
        pltpu.VMEM((K_B, CH), jnp.int32),
        pltpu.VMEM((CH, DW), _f32),
    ],
)


def _sc_agg_body(y_c, y_s, src_c3, dst_c3, src_s3, dst_s3,
                 agg_c_o, agg_s_o,
                 acc, src_v, dst_v, rows_0, rows_1, sem0, sem1):
    cid = lax.axis_index("core")
    sid = lax.axis_index("sub")

    @pl.when(cid == 0)
    def _():
        pltpu.sync_copy(y_c.at[pl.ds(sid * ROWS, ROWS)],
                        acc.at[pl.ds(sid * ROWS, ROWS)])
        pltpu.sync_copy(src_c3.at[sid], src_v)
        pltpu.sync_copy(dst_c3.at[sid], dst_v)

    @pl.when(cid == 1)
    def _():
        pltpu.sync_copy(y_s.at[pl.ds(sid * ROWS, ROWS)],
                        acc.at[pl.ds(sid * ROWS, ROWS)])
        pltpu.sync_copy(src_s3.at[sid], src_v)
        pltpu.sync_copy(dst_s3.at[sid], dst_v)

    plsc.subcore_barrier()

    def run(y_ref):
        dummy = y_ref.at[pl.ds(0, CH)]

        pltpu.async_copy(y_ref.at[src_v.at[0]], rows_0, sem0)

        def body(t, c):
            pltpu.async_copy(y_ref.at[src_v.at[2 * t + 1]], rows_1, sem1)
            pltpu.make_async_copy(dummy, rows_0, sem0).wait()
            pltpu.sync_copy(rows_0, acc.at[dst_v.at[2 * t]], add=True)
            pltpu.async_copy(y_ref.at[src_v.at[2 * t + 2]], rows_0, sem0)
            pltpu.make_async_copy(dummy, rows_1, sem1).wait()
            pltpu.sync_copy(rows_1, acc.at[dst_v.at[2 * t + 1]], add=True)
            return c

        lax.fori_loop(0, K_E // 2, body, 0)
        pltpu.make_async_copy(dummy, rows_0, sem0).wait()

    @pl.when(cid == 0)
    def _():
        run(y_c)

    @pl.when(cid == 1)
    def _():
        run(y_s)

    plsc.subcore_barrier()

    @pl.when(cid == 0)
    def _():
        pltpu.sync_copy(acc.at[pl.ds(sid * ROWS, ROWS)],
                        agg_c_o.at[pl.ds(sid * ROWS, ROWS)])

    @pl.when(cid == 1)
    def _():
        pltpu.sync_copy(acc.at[pl.ds(sid * ROWS, ROWS)],
                        agg_s_o.at[pl.ds(sid * ROWS, ROWS)])


_sc_agg = pl.kernel(
    _sc_agg_body,
    out_type=(_sds((N_P, H)), _sds((N_P, H))),
    mesh=_MESH,
    compiler_params=_SC_PARAMS,
    scratch_types=[
        pltpu.VMEM_SHARED((N_P, H), _f32),
        pltpu.VMEM((K_G, CH), jnp.int32),
        pltpu.VMEM((K_E, CH), jnp.int32),
        pltpu.VMEM((CH, H), _f32),
        pltpu.VMEM((CH, H), _f32),
        pltpu.SemaphoreType.DMA,
        pltpu.SemaphoreType.DMA,
    ],
)


def _sc_pool_body(h_c, h_s, bat_c3, bat_s3, zeros_h,
                  pool_c_o, pool_s_o,
                  accp, idx_b, rows_v):
    cid = lax.axis_index("core")
    sid = lax.axis_index("sub")

    @pl.when(sid == 0)
    def _():
        pltpu.sync_copy(zeros_h, accp)

    @pl.when(cid == 0)
    def _():
        pltpu.sync_copy(h_c.at[pl.ds(sid * ROWS, ROWS)], rows_v)
        pltpu.sync_copy(bat_c3.at[sid], idx_b)

    @pl.when(cid == 1)
    def _():
        pltpu.sync_copy(h_s.at[pl.ds(sid * ROWS, ROWS)], rows_v)
        pltpu.sync_copy(bat_s3.at[sid], idx_b)

    plsc.subcore_barrier()

    def body(j, c):
        pltpu.sync_copy(rows_v.at[pl.ds(j * CH, CH)],
                        accp.at[idx_b.at[j]], add=True)
        return c

    lax.fori_loop(0, K_B, body, 0)

    plsc.subcore_barrier()

    @pl.when((cid == 0) & (sid == 0))
    def _():
        pltpu.sync_copy(accp, pool_c_o)

    @pl.when((cid == 1) & (sid == 0))
    def _():
        pltpu.sync_copy(accp, pool_s_o)


_sc_pool = pl.kernel(
    _sc_pool_body,
    out_type=(_sds((G_P, H)), _sds((G_P, H))),
    mesh=_MESH,
    compiler_params=_SC_PARAMS,
    scratch_types=[
        pltpu.VMEM_SHARED((G_P, H), _f32),
        pltpu.VMEM((K_B, CH), jnp.int32),
        pltpu.VMEM((ROWS, H), _f32),
    ],
)


def _tc_scale_body(deg_c_r, x_c_r, w_c_r, deg_s_r, x_s_r, w_s_r,
                   y_c_o, dinv_c_o, y_s_o, dinv_s_o):
    for deg_r, x_r, w_r, y_o, dinv_o in (
            (deg_c_r, x_c_r, w_c_r, y_c_o, dinv_c_o),
            (deg_s_r, x_s_r, w_s_r, y_s_o, dinv_s_o)):
        dinv = lax.rsqrt(deg_r[:, 0:1] + 1.0)
        dinv_o[...] = dinv
        y_o[...] = jnp.dot(x_r[...], w_r[...],
                           preferred_element_type=_f32) * dinv


_tc_scale = pl.pallas_call(
    _tc_scale_body,
    out_shape=(_sds((N_P, H)), _sds((N_P, 1)), _sds((N_P, H)), _sds((N_P, 1))),
)


def _bn_relu(z, g, be):
    zr = z[:N]
    mu = jnp.mean(zr, axis=0, keepdims=True)
    var = jnp.mean((zr - mu) ** 2, axis=0, keepdims=True)
    return jnp.maximum((z - mu) * lax.rsqrt(var + EPS) * g + be, 0.0)


def _tc_mid_body(agg_c_r, dinv_c_r, g_c_r, be_c_r, w_c_r,
                 agg_s_r, dinv_s_r, g_s_r, be_s_r, w_s_r,
                 y2_c_o, y2_s_o):
    for agg_r, dinv_r, g_r, be_r, w_r, y2_o in (
            (agg_c_r, dinv_c_r, g_c_r, be_c_r, w_c_r, y2_c_o),
            (agg_s_r, dinv_s_r, g_s_r, be_s_r, w_s_r, y2_s_o)):
        dinv = dinv_r[...]
        h = _bn_relu(agg_r[...] * dinv, g_r[...], be_r[...])
        y2_o[...] = jnp.dot(h, w_r[...], preferred_element_type=_f32) * dinv


_tc_mid = pl.pallas_call(
    _tc_mid_body,
    out_shape=(_sds((N_P, H)), _sds((N_P, H))),
)


def _tc_h2_body(agg_c_r, dinv_c_r, g_c_r, be_c_r,
                agg_s_r, dinv_s_r, g_s_r, be_s_r,
                h_c_o, h_s_o):
    h_c_o[...] = _bn_relu(agg_c_r[...] * dinv_c_r[...], g_c_r[...], be_c_r[...])
    h_s_o[...] = _bn_relu(agg_s_r[...] * dinv_s_r[...], g_s_r[...], be_s_r[...])


_tc_h2 = pl.pallas_call(
    _tc_h2_body,
    out_shape=(_sds((N_P, H)), _sds((N_P, H))),
)


def _tc_head_body(pool_c_r, cnt_c_r, pool_s_r, cnt_s_r,
                  wf1_r, bf1_r, wf2_r, bf2_r, out_o):
    p_c = pool_c_r[0:G] / jnp.maximum(cnt_c_r[0:G, 0:1], 1.0)
    p_s = pool_s_r[0:G] / jnp.maximum(cnt_s_r[0:G, 0:1], 1.0)
    xcat = jnp.concatenate([p_c, p_s], axis=1)
    hh = jnp.maximum(
        jnp.dot(xcat, wf1_r[...], preferred_element_type=_f32) + bf1_r[...],
        0.0)
    out_o[...] = jnp.dot(hh, wf2_r[...], preferred_element_type=_f32) + bf2_r[...]


_tc_head = pl.pallas_call(
    _tc_head_body,
    out_shape=_sds((G, OUT)),
)


def kernel(x_c, edge_index_c, batch_c, x_s, edge_index_s, batch_s,
           W1_c, b1_c, g1_c, be1_c, W2_c, b2_c, g2_c, be2_c,
           W1_s, b1_s, g1_s, be1_s, W2_s, b2_s, g2_s, be2_s,
           Wf1, bf1, Wf2, bf2):
    x_cp = jnp.pad(x_c, ((0, N_P - N), (0, 0)))
    x_sp = jnp.pad(x_s, ((0, N_P - N), (0, 0)))

    def edges3(ei):
        src = jnp.pad(ei[0], (0, E_P - E)).reshape(NS, K_E, CH)
        src = jnp.pad(src, ((0, 0), (0, K_G - K_E), (0, 0)))
        dst = jnp.pad(ei[1], (0, E_P - E), constant_values=N).reshape(NS, K_E, CH)
        return src, dst

    src_c3, dst_c3 = edges3(edge_index_c)
    src_s3, dst_s3 = edges3(edge_index_s)
    bat_c3 = jnp.pad(batch_c, (0, N_P - N), constant_values=G).reshape(NS, K_B, CH)
    bat_s3 = jnp.pad(batch_s, (0, N_P - N), constant_values=G).reshape(NS, K_B, CH)

    ones_h = jnp.ones((CH, DW), _f32)
    zeros_deg = jnp.zeros((N_P, DW), _f32)
    zeros_pool = jnp.zeros((G_P, H), _f32)

    deg_c, deg_s, cnt_c, cnt_s = _sc_degcnt(
        dst_c3, dst_s3, bat_c3, bat_s3, ones_h, zeros_deg)

    y1_c, dinv_c, y1_s, dinv_s = _tc_scale(
        deg_c, x_cp, W1_c, deg_s, x_sp, W1_s)

    agg1_c, agg1_s = _sc_agg(y1_c, y1_s, src_c3, dst_c3, src_s3, dst_s3)

    g1_c2, be1_c2 = g1_c.reshape(1, H), be1_c.reshape(1, H)
    g1_s2, be1_s2 = g1_s.reshape(1, H), be1_s.reshape(1, H)
    g2_c2, be2_c2 = g2_c.reshape(1, H), be2_c.reshape(1, H)
    g2_s2, be2_s2 = g2_s.reshape(1, H), be2_s.reshape(1, H)

    y2_c, y2_s = _tc_mid(
        agg1_c, dinv_c, g1_c2, be1_c2, W2_c,
        agg1_s, dinv_s, g1_s2, be1_s2, W2_s)

    agg2_c, agg2_s = _sc_agg(y2_c, y2_s, src_c3, dst_c3, src_s3, dst_s3)

    h2_c, h2_s = _tc_h2(
        agg2_c, dinv_c, g2_c2, be2_c2,
        agg2_s, dinv_s, g2_s2, be2_s2)

    pool_c, pool_s = _sc_pool(h2_c, h2_s, bat_c3, bat_s3, zeros_pool)

    return _tc_head(pool_c, cnt_c, pool_s, cnt_s,
                    Wf1, bf1.reshape(1, H), Wf2, bf2.reshape(1, OUT))
